# baseline (speedup 1.0000x reference)
import functools

import jax
import jax.numpy as jnp
from jax import lax
from jax.experimental import pallas as pl
from jax.experimental.pallas import tpu as pltpu
from jax.experimental.pallas import tpu_sc as plsc

NC = 2
NS = 16
NW = NC * NS
CH = 128
N = 10000
AR = 10240
RPT = AR // NS
ZI = RPT // CH


def _make_agg(F, nproc, ch, nb):
    nslab = nproc + nb
    zi = RPT // ch
    mesh = plsc.VectorSubcoreMesh(core_axis_name="c", subcore_axis_name="s")

    @functools.partial(
        pl.kernel,
        out_type=jax.ShapeDtypeStruct((NC, AR, F), jnp.float32),
        mesh=mesh,
        compiler_params=pltpu.CompilerParams(use_tc_tiling_on_sc=False),
        scratch_types=[
            pltpu.VMEM((nslab, ch), jnp.int32),
            pltpu.VMEM((nslab, ch), jnp.int32),
            pltpu.VMEM((nb, ch, F), jnp.float32),
            pltpu.VMEM_SHARED((AR, F), jnp.float32),
            [pltpu.SemaphoreType.DMA] * nb,
        ],
    )
    def agg(g_hbm, src_hbm, dst_hbm, out_hbm, src_v, dst_v, rows_v, acc,
            sems):
        cid = lax.axis_index("c")
        sid = lax.axis_index("s")
        wid = sid * NC + cid
        pltpu.sync_copy(src_hbm.at[wid], src_v)
        pltpu.sync_copy(dst_hbm.at[wid], dst_v)

        zvec = jnp.zeros((16,), jnp.float32)

        def zrow(i, carry):
            for jj in range(F // 16):
                rows_v[0, i, pl.ds(jj * 16, 16)] = zvec
            return carry

        lax.fori_loop(0, ch, zrow, 0)
        r0 = sid * RPT
        for z in range(zi):
            pltpu.sync_copy(rows_v.at[0], acc.at[pl.ds(r0 + z * ch, ch)])

        plsc.subcore_barrier()
        for b in range(nb):
            pltpu.async_copy(g_hbm.at[src_v.at[b]], rows_v.at[b], sems[b])

        def body(q, carry):
            j0 = q * nb
            for b in range(nb):
                pltpu.make_async_copy(
                    g_hbm.at[src_v.at[0]], rows_v.at[b], sems[b]).wait()
                pltpu.sync_copy(rows_v.at[b], acc.at[dst_v.at[j0 + b]],
                                add=True)
                pltpu.async_copy(g_hbm.at[src_v.at[j0 + b + nb]],
                                 rows_v.at[b], sems[b])
            return carry

        lax.fori_loop(0, nproc // nb, body, 0)
        for b in range(nb):
            pltpu.make_async_copy(
                g_hbm.at[src_v.at[0]], rows_v.at[b], sems[b]).wait()
        plsc.subcore_barrier()
        pltpu.sync_copy(acc.at[pl.ds(r0, RPT)],
                        out_hbm.at[cid, pl.ds(r0, RPT)])

    return agg


def _make_deg(nproc, ch, nb):
    F = 16
    nslab = nproc + nb
    zi = RPT // ch
    mesh = plsc.VectorSubcoreMesh(core_axis_name="c", subcore_axis_name="s")

    @functools.partial(
        pl.kernel,
        out_type=jax.ShapeDtypeStruct((NC, AR, F), jnp.float32),
        mesh=mesh,
        compiler_params=pltpu.CompilerParams(use_tc_tiling_on_sc=False),
        scratch_types=[
            pltpu.VMEM((nslab, ch), jnp.int32),
            pltpu.VMEM((ch, F), jnp.float32),
            pltpu.VMEM_SHARED((AR, F), jnp.float32),
        ],
    )
    def deg(dst_hbm, out_hbm, dst_v, rows_v, acc):
        cid = lax.axis_index("c")
        sid = lax.axis_index("s")
        wid = sid * NC + cid
        pltpu.sync_copy(dst_hbm.at[wid], dst_v)

        zvec = jnp.zeros((16,), jnp.float32)

        def zrow(i, carry):
            rows_v[i, pl.ds(0, 16)] = zvec
            return carry

        lax.fori_loop(0, ch, zrow, 0)
        r0 = sid * RPT
        for z in range(zi):
            pltpu.sync_copy(rows_v, acc.at[pl.ds(r0 + z * ch, ch)])
        plsc.subcore_barrier()

        ovec = jnp.full((16,), 1.0, jnp.float32)

        def orow(i, carry):
            rows_v[i, pl.ds(0, 16)] = ovec
            return carry

        lax.fori_loop(0, ch, orow, 0)

        def body(j, carry):
            pltpu.sync_copy(rows_v, acc.at[dst_v.at[j]], add=True)
            return carry

        lax.fori_loop(0, nproc, body, 0)
        plsc.subcore_barrier()
        pltpu.sync_copy(acc.at[pl.ds(r0, RPT)],
                        out_hbm.at[cid, pl.ds(r0, RPT)])

    return deg


def _dinv_from_deg(deg_acc):
    def body(deg_ref, out_ref):
        d = deg_ref[0, :, 0:1] + deg_ref[1, :, 0:1] + 1.0
        out_ref[...] = lax.rsqrt(jnp.maximum(d, 1e-12))

    return pl.pallas_call(
        body,
        out_shape=jax.ShapeDtypeStruct((AR, 1), jnp.float32),
    )(deg_acc)


def _tc_stage(g, dinv, acc=None, in_scale=True, b_pre=None, relu_pre=False,
              Wa=None, ba=None, relu_a=False, Wb=None, out_scale=True,
              R=1000):
    Fin = g.shape[1]
    Fout = Wb.shape[1] if Wb is not None else (
        Wa.shape[1] if Wa is not None else Fin)

    operands = []
    specs = []
    flags = dict(has_acc=acc is not None, has_bpre=b_pre is not None,
                 has_wa=Wa is not None, has_ba=ba is not None,
                 has_wb=Wb is not None)
    if acc is not None:
        operands.append(acc)
        specs.append(pl.BlockSpec((2, R, Fin), lambda i: (0, i, 0)))
    operands.append(g)
    specs.append(pl.BlockSpec((R, Fin), lambda i: (i, 0)))
    operands.append(dinv)
    specs.append(pl.BlockSpec((R, 1), lambda i: (i, 0)))
    if b_pre is not None:
        operands.append(b_pre.reshape(1, -1))
        specs.append(pl.BlockSpec((1, Fin), lambda i: (0, 0)))
    if Wa is not None:
        operands.append(Wa)
        specs.append(pl.BlockSpec(Wa.shape, lambda i: (0, 0)))
    if ba is not None:
        operands.append(ba.reshape(1, -1))
        specs.append(pl.BlockSpec((1, ba.shape[0]), lambda i: (0, 0)))
    if Wb is not None:
        operands.append(Wb)
        specs.append(pl.BlockSpec(Wb.shape, lambda i: (0, 0)))

    def body(*refs):
        it = iter(refs)
        acc_ref = next(it) if flags["has_acc"] else None
        g_ref = next(it)
        dinv_ref = next(it)
        bpre_ref = next(it) if flags["has_bpre"] else None
        wa_ref = next(it) if flags["has_wa"] else None
        ba_ref = next(it) if flags["has_ba"] else None
        wb_ref = next(it) if flags["has_wb"] else None
        out_ref = next(it)

        t = g_ref[...]
        if acc_ref is not None:
            t = t + acc_ref[0] + acc_ref[1]
        dv = dinv_ref[...]
        if in_scale:
            t = t * dv
        if bpre_ref is not None:
            t = t + bpre_ref[...]
            if relu_pre:
                t = jnp.maximum(t, 0.0)
        if wa_ref is not None:
            t = jnp.dot(t, wa_ref[...], preferred_element_type=jnp.float32)
            if ba_ref is not None:
                t = t + ba_ref[...]
            if relu_a:
                t = jnp.maximum(t, 0.0)
        if wb_ref is not None:
            t = jnp.dot(t, wb_ref[...], preferred_element_type=jnp.float32)
        if out_scale:
            t = t * dv
        out_ref[...] = t

    return pl.pallas_call(
        body,
        grid=(N // R,),
        in_specs=specs,
        out_specs=pl.BlockSpec((R, Fout), lambda i: (i, 0)),
        out_shape=jax.ShapeDtypeStruct((N, Fout), jnp.float32),
    )(*operands)


def _agg_call(g, slabs):
    F = g.shape[1]
    if F <= 64:
        s, d, npc, nb = slabs[128]
        return _make_agg(F, npc, 128, nb)(g, s, d)
    s, d, npc, nb = slabs[64]
    if F == 128:
        return _make_agg(128, npc, 64, nb)(g, s, d)
    parts = [
        _make_agg(128, npc, 64, nb)(
            lax.slice_in_dim(g, c * 128, (c + 1) * 128, axis=1), s, d)
        for c in range(F // 128)
    ]
    return jnp.concatenate(parts, axis=2)


def kernel(x, edge_index, W0, b0, W1, b1, W2, b2, W3, b3, W4, b4, W5, b5,
           W6, b6, W7, b7):
    E = edge_index.shape[1]

    def build_slabs(ch, nb):
        npc = -(-E // (NW * ch))
        npc = -(-npc // nb) * nb
        ep = NW * npc * ch
        src_p = jnp.concatenate(
            [edge_index[0], jnp.zeros((ep - E,), jnp.int32)])
        dst_p = jnp.concatenate(
            [edge_index[1], jnp.full((ep - E,), N, jnp.int32)])
        s = jnp.concatenate(
            [src_p.reshape(NW, npc, ch),
             jnp.zeros((NW, nb, ch), jnp.int32)], axis=1)
        d = jnp.concatenate(
            [dst_p.reshape(NW, npc, ch),
             jnp.full((NW, nb, ch), N, jnp.int32)], axis=1)
        return s, d, npc, nb

    slabs = {128: build_slabs(128, 4), 64: build_slabs(64, 3)}

    deg_acc = _make_deg(slabs[128][2], 128, 4)(slabs[128][1])
    dinv = _dinv_from_deg(deg_acc)

    g = _tc_stage(x, dinv, in_scale=False, Wa=W0)
    acc = _agg_call(g, slabs)
    for b_i, W_next in ((b0, W1), (b1, W2)):
        g = _tc_stage(g, dinv, acc=acc, b_pre=b_i, relu_pre=True, Wa=W_next)
        acc = _agg_call(g, slabs)
    g = _tc_stage(g, dinv, acc=acc, b_pre=b2, relu_pre=True)
    acc = _agg_call(g, slabs)
    g = _tc_stage(g, dinv, acc=acc, Wa=W3, ba=b3, relu_a=True)
    acc = _agg_call(g, slabs)
    g = _tc_stage(g, dinv, acc=acc, Wa=W4, ba=b4, relu_a=True, Wb=W5)
    acc = _agg_call(g, slabs)
    for b_i, W_next in ((b5, W6), (b6, W7)):
        g = _tc_stage(g, dinv, acc=acc, b_pre=b_i, relu_pre=True, Wa=W_next)
        acc = _agg_call(g, slabs)
    return _tc_stage(g, dinv, acc=acc, b_pre=b7, in_scale=True,
                     out_scale=False)


# speedup vs baseline: 1.1220x; 1.1220x over previous
import functools

import jax
import jax.numpy as jnp
from jax import lax
from jax.experimental import pallas as pl
from jax.experimental.pallas import tpu as pltpu
from jax.experimental.pallas import tpu_sc as plsc

NC = 2
NS = 16
NW = NC * NS
CH = 128
N = 10000
AR = 10240
RPT = AR // NS
ZI = RPT // CH


def _make_agg(F, ngroups, ch, k):
    zi = RPT // ch
    mesh = plsc.VectorSubcoreMesh(core_axis_name="c", subcore_axis_name="s")

    @functools.partial(
        pl.kernel,
        out_type=jax.ShapeDtypeStruct((NC, AR, F), jnp.float32),
        mesh=mesh,
        compiler_params=pltpu.CompilerParams(use_tc_tiling_on_sc=False),
        scratch_types=[
            pltpu.VMEM((2, k, ch), jnp.int32),
            pltpu.VMEM((2, k, ch), jnp.int32),
            pltpu.VMEM((2, k, ch, F), jnp.float32),
            pltpu.VMEM_SHARED((AR, F), jnp.float32),
            [pltpu.SemaphoreType.DMA] * 2,
            [pltpu.SemaphoreType.DMA] * 2,
            [pltpu.SemaphoreType.DMA] * 2,
        ],
    )
    def agg(g_hbm, src_hbm, dst_hbm, out_hbm, src_v, dst_v, rows_v, acc,
            isem, gsem, ssem):
        cid = lax.axis_index("c")
        sid = lax.axis_index("s")
        wid = sid * NC + cid

        zvec = jnp.zeros((16,), jnp.float32)

        def zrow(i, carry):
            for jj in range(F // 16):
                rows_v[0, 0, i, pl.ds(jj * 16, 16)] = zvec
            return carry

        lax.fori_loop(0, ch, zrow, 0)
        r0 = sid * RPT
        for z in range(zi):
            pltpu.sync_copy(rows_v.at[0, 0], acc.at[pl.ds(r0 + z * ch, ch)])

        pltpu.sync_copy(src_hbm.at[wid, 0], src_v.at[0])
        pltpu.sync_copy(dst_hbm.at[wid, 0], dst_v.at[0])
        for b in range(k):
            pltpu.async_copy(g_hbm.at[src_v.at[0, b]], rows_v.at[0, b],
                             gsem[0])
        pltpu.async_copy(src_hbm.at[wid, 1], src_v.at[1], isem[1])
        pltpu.async_copy(dst_hbm.at[wid, 1], dst_v.at[1], isem[1])
        plsc.subcore_barrier()

        def step(g, p, q):
            pltpu.make_async_copy(src_hbm.at[wid, 0], src_v.at[q],
                                  isem[q]).wait()
            pltpu.make_async_copy(dst_hbm.at[wid, 0], dst_v.at[q],
                                  isem[q]).wait()
            for b in range(k):
                pltpu.async_copy(g_hbm.at[src_v.at[q, b]],
                                 rows_v.at[q, b], gsem[q])
            for b in range(k):
                pltpu.make_async_copy(g_hbm.at[src_v.at[p, b]],
                                      rows_v.at[p, b], gsem[p]).wait()
            for b in range(k):
                pltpu.async_copy(rows_v.at[p, b], acc.at[dst_v.at[p, b]],
                                 ssem[p], add=True)
            for b in range(k):
                pltpu.make_async_copy(rows_v.at[p, b],
                                      acc.at[dst_v.at[p, b]],
                                      ssem[p]).wait()
            pltpu.async_copy(src_hbm.at[wid, g + 2], src_v.at[p], isem[p])
            pltpu.async_copy(dst_hbm.at[wid, g + 2], dst_v.at[p], isem[p])

        def body(t, carry):
            step(2 * t, 0, 1)
            step(2 * t + 1, 1, 0)
            return carry

        lax.fori_loop(0, ngroups // 2, body, 0)
        for b in range(k):
            pltpu.make_async_copy(g_hbm.at[src_v.at[0, b]],
                                  rows_v.at[0, b], gsem[0]).wait()
        pltpu.make_async_copy(src_hbm.at[wid, 0], src_v.at[1],
                              isem[1]).wait()
        pltpu.make_async_copy(dst_hbm.at[wid, 0], dst_v.at[1],
                              isem[1]).wait()
        plsc.subcore_barrier()
        pltpu.sync_copy(acc.at[pl.ds(r0, RPT)],
                        out_hbm.at[cid, pl.ds(r0, RPT)])

    return agg


def _make_deg(nproc, ch, nb):
    F = 16
    nslab = nproc + nb
    zi = RPT // ch
    mesh = plsc.VectorSubcoreMesh(core_axis_name="c", subcore_axis_name="s")

    @functools.partial(
        pl.kernel,
        out_type=jax.ShapeDtypeStruct((NC, AR, F), jnp.float32),
        mesh=mesh,
        compiler_params=pltpu.CompilerParams(use_tc_tiling_on_sc=False),
        scratch_types=[
            pltpu.VMEM((nslab, ch), jnp.int32),
            pltpu.VMEM((ch, F), jnp.float32),
            pltpu.VMEM_SHARED((AR, F), jnp.float32),
        ],
    )
    def deg(dst_hbm, out_hbm, dst_v, rows_v, acc):
        cid = lax.axis_index("c")
        sid = lax.axis_index("s")
        wid = sid * NC + cid
        pltpu.sync_copy(dst_hbm.at[wid], dst_v)

        zvec = jnp.zeros((16,), jnp.float32)

        def zrow(i, carry):
            rows_v[i, pl.ds(0, 16)] = zvec
            return carry

        lax.fori_loop(0, ch, zrow, 0)
        r0 = sid * RPT
        for z in range(zi):
            pltpu.sync_copy(rows_v, acc.at[pl.ds(r0 + z * ch, ch)])
        plsc.subcore_barrier()

        ovec = jnp.full((16,), 1.0, jnp.float32)

        def orow(i, carry):
            rows_v[i, pl.ds(0, 16)] = ovec
            return carry

        lax.fori_loop(0, ch, orow, 0)

        def body(j, carry):
            pltpu.sync_copy(rows_v, acc.at[dst_v.at[j]], add=True)
            return carry

        lax.fori_loop(0, nproc, body, 0)
        plsc.subcore_barrier()
        pltpu.sync_copy(acc.at[pl.ds(r0, RPT)],
                        out_hbm.at[cid, pl.ds(r0, RPT)])

    return deg


def _dinv_from_deg(deg_acc):
    def body(deg_ref, out_ref):
        d = deg_ref[0, :, 0:1] + deg_ref[1, :, 0:1] + 1.0
        out_ref[...] = lax.rsqrt(jnp.maximum(d, 1e-12))

    return pl.pallas_call(
        body,
        out_shape=jax.ShapeDtypeStruct((AR, 1), jnp.float32),
    )(deg_acc)


def _tc_stage(g, dinv, acc=None, in_scale=True, b_pre=None, relu_pre=False,
              Wa=None, ba=None, relu_a=False, Wb=None, out_scale=True,
              R=1000):
    Fin = g.shape[1]
    Fout = Wb.shape[1] if Wb is not None else (
        Wa.shape[1] if Wa is not None else Fin)

    operands = []
    specs = []
    flags = dict(has_acc=acc is not None, has_bpre=b_pre is not None,
                 has_wa=Wa is not None, has_ba=ba is not None,
                 has_wb=Wb is not None)
    if acc is not None:
        operands.append(acc)
        specs.append(pl.BlockSpec((2, R, Fin), lambda i: (0, i, 0)))
    operands.append(g)
    specs.append(pl.BlockSpec((R, Fin), lambda i: (i, 0)))
    operands.append(dinv)
    specs.append(pl.BlockSpec((R, 1), lambda i: (i, 0)))
    if b_pre is not None:
        operands.append(b_pre.reshape(1, -1))
        specs.append(pl.BlockSpec((1, Fin), lambda i: (0, 0)))
    if Wa is not None:
        operands.append(Wa)
        specs.append(pl.BlockSpec(Wa.shape, lambda i: (0, 0)))
    if ba is not None:
        operands.append(ba.reshape(1, -1))
        specs.append(pl.BlockSpec((1, ba.shape[0]), lambda i: (0, 0)))
    if Wb is not None:
        operands.append(Wb)
        specs.append(pl.BlockSpec(Wb.shape, lambda i: (0, 0)))

    def body(*refs):
        it = iter(refs)
        acc_ref = next(it) if flags["has_acc"] else None
        g_ref = next(it)
        dinv_ref = next(it)
        bpre_ref = next(it) if flags["has_bpre"] else None
        wa_ref = next(it) if flags["has_wa"] else None
        ba_ref = next(it) if flags["has_ba"] else None
        wb_ref = next(it) if flags["has_wb"] else None
        out_ref = next(it)

        t = g_ref[...]
        if acc_ref is not None:
            t = t + acc_ref[0] + acc_ref[1]
        dv = dinv_ref[...]
        if in_scale:
            t = t * dv
        if bpre_ref is not None:
            t = t + bpre_ref[...]
            if relu_pre:
                t = jnp.maximum(t, 0.0)
        if wa_ref is not None:
            t = jnp.dot(t, wa_ref[...], preferred_element_type=jnp.float32)
            if ba_ref is not None:
                t = t + ba_ref[...]
            if relu_a:
                t = jnp.maximum(t, 0.0)
        if wb_ref is not None:
            t = jnp.dot(t, wb_ref[...], preferred_element_type=jnp.float32)
        if out_scale:
            t = t * dv
        out_ref[...] = t

    return pl.pallas_call(
        body,
        grid=(N // R,),
        in_specs=specs,
        out_specs=pl.BlockSpec((R, Fout), lambda i: (i, 0)),
        out_shape=jax.ShapeDtypeStruct((N, Fout), jnp.float32),
    )(*operands)


def _agg_call(g, slabs):
    F = g.shape[1]
    if F <= 64:
        s, d, ng, k = slabs[128]
        return _make_agg(F, ng, 128, k)(g, s, d)
    s, d, ng, k = slabs[64]
    if F == 128:
        return _make_agg(128, ng, 64, k)(g, s, d)
    parts = [
        _make_agg(128, ng, 64, k)(
            lax.slice_in_dim(g, c * 128, (c + 1) * 128, axis=1), s, d)
        for c in range(F // 128)
    ]
    return jnp.concatenate(parts, axis=2)


def kernel(x, edge_index, W0, b0, W1, b1, W2, b2, W3, b3, W4, b4, W5, b5,
           W6, b6, W7, b7):
    E = edge_index.shape[1]

    def build_slabs(ch, k):
        npc = -(-E // (NW * ch))
        npc = -(-npc // (2 * k)) * (2 * k)
        ng = npc // k
        ep = NW * npc * ch
        src_p = jnp.concatenate(
            [edge_index[0], jnp.zeros((ep - E,), jnp.int32)])
        dst_p = jnp.concatenate(
            [edge_index[1], jnp.full((ep - E,), N, jnp.int32)])
        s = jnp.concatenate(
            [src_p.reshape(NW, ng, k, ch),
             jnp.zeros((NW, 2, k, ch), jnp.int32)], axis=1)
        d = jnp.concatenate(
            [dst_p.reshape(NW, ng, k, ch),
             jnp.full((NW, 2, k, ch), N, jnp.int32)], axis=1)
        return s, d, ng, k

    slabs = {128: build_slabs(128, 2), 64: build_slabs(64, 2)}

    d128 = slabs[128][1]
    deg_acc = _make_deg(d128.shape[1] * d128.shape[2], 128, 0)(
        d128.reshape(NW, -1, 128))
    dinv = _dinv_from_deg(deg_acc)

    g = _tc_stage(x, dinv, in_scale=False, Wa=W0)
    acc = _agg_call(g, slabs)
    for b_i, W_next in ((b0, W1), (b1, W2)):
        g = _tc_stage(g, dinv, acc=acc, b_pre=b_i, relu_pre=True, Wa=W_next)
        acc = _agg_call(g, slabs)
    g = _tc_stage(g, dinv, acc=acc, b_pre=b2, relu_pre=True)
    acc = _agg_call(g, slabs)
    g = _tc_stage(g, dinv, acc=acc, Wa=W3, ba=b3, relu_a=True)
    acc = _agg_call(g, slabs)
    g = _tc_stage(g, dinv, acc=acc, Wa=W4, ba=b4, relu_a=True, Wb=W5)
    acc = _agg_call(g, slabs)
    for b_i, W_next in ((b5, W6), (b6, W7)):
        g = _tc_stage(g, dinv, acc=acc, b_pre=b_i, relu_pre=True, Wa=W_next)
        acc = _agg_call(g, slabs)
    return _tc_stage(g, dinv, acc=acc, b_pre=b7, in_scale=True,
                     out_scale=False)


# speedup vs baseline: 1.3230x; 1.1791x over previous
import functools

import jax
import jax.numpy as jnp
from jax import lax
from jax.experimental import pallas as pl
from jax.experimental.pallas import tpu as pltpu
from jax.experimental.pallas import tpu_sc as plsc

NC = 2
NS = 16
NW = NC * NS
CH = 128
N = 10000
AR = 10240
RPT = AR // NS
ZI = RPT // CH


def _make_agg(F, nproc, sg):
    zi = RPT // CH
    nsup = nproc // sg
    mesh = plsc.VectorSubcoreMesh(core_axis_name="c", subcore_axis_name="s")

    @functools.partial(
        pl.kernel,
        out_type=jax.ShapeDtypeStruct((NC, AR, F), jnp.float32),
        mesh=mesh,
        compiler_params=pltpu.CompilerParams(use_tc_tiling_on_sc=False),
        scratch_types=[
            pltpu.VMEM((nsup, sg * CH), jnp.int32),
            pltpu.VMEM((nproc, CH), jnp.int32),
            pltpu.VMEM((sg * CH, F), jnp.float32),
            pltpu.VMEM_SHARED((AR, F), jnp.float32),
            pltpu.SemaphoreType.DMA,
        ],
    )
    def agg(g_hbm, src_hbm, dst_hbm, out_hbm, src_v, dst_v, rows_v, acc,
            sem):
        cid = lax.axis_index("c")
        sid = lax.axis_index("s")
        wid = sid * NC + cid
        pltpu.sync_copy(src_hbm.at[wid], src_v)
        pltpu.sync_copy(dst_hbm.at[wid], dst_v)

        zvec = jnp.zeros((16,), jnp.float32)

        def zrow(i, carry):
            for jj in range(F // 16):
                rows_v[i, pl.ds(jj * 16, 16)] = zvec
            return carry

        lax.fori_loop(0, CH, zrow, 0)
        r0 = sid * RPT
        for z in range(zi):
            pltpu.sync_copy(rows_v.at[pl.ds(0, CH)],
                            acc.at[pl.ds(r0 + z * CH, CH)])
        plsc.subcore_barrier()

        def body(s, carry):
            pltpu.async_copy(g_hbm.at[src_v.at[s]], rows_v, sem).wait()
            for b in range(sg):
                pltpu.sync_copy(rows_v.at[pl.ds(b * CH, CH)],
                                acc.at[dst_v.at[s * sg + b]], add=True)
            return carry

        lax.fori_loop(0, nsup, body, 0)
        plsc.subcore_barrier()
        pltpu.sync_copy(acc.at[pl.ds(r0, RPT)],
                        out_hbm.at[cid, pl.ds(r0, RPT)])

    return agg


def _make_deg(nproc, ch, nb):
    F = 16
    nslab = nproc + nb
    zi = RPT // ch
    mesh = plsc.VectorSubcoreMesh(core_axis_name="c", subcore_axis_name="s")

    @functools.partial(
        pl.kernel,
        out_type=jax.ShapeDtypeStruct((NC, AR, F), jnp.float32),
        mesh=mesh,
        compiler_params=pltpu.CompilerParams(use_tc_tiling_on_sc=False),
        scratch_types=[
            pltpu.VMEM((nslab, ch), jnp.int32),
            pltpu.VMEM((ch, F), jnp.float32),
            pltpu.VMEM_SHARED((AR, F), jnp.float32),
        ],
    )
    def deg(dst_hbm, out_hbm, dst_v, rows_v, acc):
        cid = lax.axis_index("c")
        sid = lax.axis_index("s")
        wid = sid * NC + cid
        pltpu.sync_copy(dst_hbm.at[wid], dst_v)

        zvec = jnp.zeros((16,), jnp.float32)

        def zrow(i, carry):
            rows_v[i, pl.ds(0, 16)] = zvec
            return carry

        lax.fori_loop(0, ch, zrow, 0)
        r0 = sid * RPT
        for z in range(zi):
            pltpu.sync_copy(rows_v, acc.at[pl.ds(r0 + z * ch, ch)])
        plsc.subcore_barrier()

        ovec = jnp.full((16,), 1.0, jnp.float32)

        def orow(i, carry):
            rows_v[i, pl.ds(0, 16)] = ovec
            return carry

        lax.fori_loop(0, ch, orow, 0)

        def body(j, carry):
            pltpu.sync_copy(rows_v, acc.at[dst_v.at[j]], add=True)
            return carry

        lax.fori_loop(0, nproc, body, 0)
        plsc.subcore_barrier()
        pltpu.sync_copy(acc.at[pl.ds(r0, RPT)],
                        out_hbm.at[cid, pl.ds(r0, RPT)])

    return deg


def _dinv_from_deg(deg_acc):
    def body(deg_ref, out_ref):
        d = deg_ref[0, :, 0:1] + deg_ref[1, :, 0:1] + 1.0
        out_ref[...] = lax.rsqrt(jnp.maximum(d, 1e-12))

    return pl.pallas_call(
        body,
        out_shape=jax.ShapeDtypeStruct((AR, 1), jnp.float32),
    )(deg_acc)


def _tc_stage(g, dinv, acc=None, in_scale=True, b_pre=None, relu_pre=False,
              Wa=None, ba=None, relu_a=False, Wb=None, out_scale=True,
              R=1000):
    Fin = g.shape[1]
    Fout = Wb.shape[1] if Wb is not None else (
        Wa.shape[1] if Wa is not None else Fin)

    operands = []
    specs = []
    flags = dict(has_acc=acc is not None, has_bpre=b_pre is not None,
                 has_wa=Wa is not None, has_ba=ba is not None,
                 has_wb=Wb is not None)
    if acc is not None:
        operands.append(acc)
        specs.append(pl.BlockSpec((2, R, Fin), lambda i: (0, i, 0)))
    operands.append(g)
    specs.append(pl.BlockSpec((R, Fin), lambda i: (i, 0)))
    operands.append(dinv)
    specs.append(pl.BlockSpec((R, 1), lambda i: (i, 0)))
    if b_pre is not None:
        operands.append(b_pre.reshape(1, -1))
        specs.append(pl.BlockSpec((1, Fin), lambda i: (0, 0)))
    if Wa is not None:
        operands.append(Wa)
        specs.append(pl.BlockSpec(Wa.shape, lambda i: (0, 0)))
    if ba is not None:
        operands.append(ba.reshape(1, -1))
        specs.append(pl.BlockSpec((1, ba.shape[0]), lambda i: (0, 0)))
    if Wb is not None:
        operands.append(Wb)
        specs.append(pl.BlockSpec(Wb.shape, lambda i: (0, 0)))

    def body(*refs):
        it = iter(refs)
        acc_ref = next(it) if flags["has_acc"] else None
        g_ref = next(it)
        dinv_ref = next(it)
        bpre_ref = next(it) if flags["has_bpre"] else None
        wa_ref = next(it) if flags["has_wa"] else None
        ba_ref = next(it) if flags["has_ba"] else None
        wb_ref = next(it) if flags["has_wb"] else None
        out_ref = next(it)

        t = g_ref[...]
        if acc_ref is not None:
            t = t + acc_ref[0] + acc_ref[1]
        dv = dinv_ref[...]
        if in_scale:
            t = t * dv
        if bpre_ref is not None:
            t = t + bpre_ref[...]
            if relu_pre:
                t = jnp.maximum(t, 0.0)
        if wa_ref is not None:
            t = jnp.dot(t, wa_ref[...], preferred_element_type=jnp.float32)
            if ba_ref is not None:
                t = t + ba_ref[...]
            if relu_a:
                t = jnp.maximum(t, 0.0)
        if wb_ref is not None:
            t = jnp.dot(t, wb_ref[...], preferred_element_type=jnp.float32)
        if out_scale:
            t = t * dv
        out_ref[...] = t

    return pl.pallas_call(
        body,
        grid=(N // R,),
        in_specs=specs,
        out_specs=pl.BlockSpec((R, Fout), lambda i: (i, 0)),
        out_shape=jax.ShapeDtypeStruct((N, Fout), jnp.float32),
    )(*operands)


def _agg_call(g, slabs):
    src4, src1, dst, npc = slabs
    F = g.shape[1]
    if F <= 64:
        return _make_agg(F, npc, 4)(g, src4, dst)
    if F == 128:
        return _make_agg(128, npc, 1)(g, src1, dst)
    parts = [
        _make_agg(128, npc, 1)(
            lax.slice_in_dim(g, c * 128, (c + 1) * 128, axis=1),
            src1, dst)
        for c in range(F // 128)
    ]
    return jnp.concatenate(parts, axis=2)


def kernel(x, edge_index, W0, b0, W1, b1, W2, b2, W3, b3, W4, b4, W5, b5,
           W6, b6, W7, b7):
    E = edge_index.shape[1]

    npc = -(-E // (NW * CH))
    npc = -(-npc // 4) * 4
    ep = NW * npc * CH
    src_p = jnp.concatenate(
        [edge_index[0], jnp.zeros((ep - E,), jnp.int32)])
    dst_p = jnp.concatenate(
        [edge_index[1], jnp.full((ep - E,), N, jnp.int32)])
    slabs = (src_p.reshape(NW, npc // 4, 4 * CH),
             src_p.reshape(NW, npc, CH),
             dst_p.reshape(NW, npc, CH),
             npc)

    deg_acc = _make_deg(npc, CH, 0)(slabs[2])
    dinv = _dinv_from_deg(deg_acc)

    g = _tc_stage(x, dinv, in_scale=False, Wa=W0)
    acc = _agg_call(g, slabs)
    for b_i, W_next in ((b0, W1), (b1, W2)):
        g = _tc_stage(g, dinv, acc=acc, b_pre=b_i, relu_pre=True, Wa=W_next)
        acc = _agg_call(g, slabs)
    g = _tc_stage(g, dinv, acc=acc, b_pre=b2, relu_pre=True)
    acc = _agg_call(g, slabs)
    g = _tc_stage(g, dinv, acc=acc, Wa=W3, ba=b3, relu_a=True)
    acc = _agg_call(g, slabs)
    g = _tc_stage(g, dinv, acc=acc, Wa=W4, ba=b4, relu_a=True, Wb=W5)
    acc = _agg_call(g, slabs)
    for b_i, W_next in ((b5, W6), (b6, W7)):
        g = _tc_stage(g, dinv, acc=acc, b_pre=b_i, relu_pre=True, Wa=W_next)
        acc = _agg_call(g, slabs)
    return _tc_stage(g, dinv, acc=acc, b_pre=b7, in_scale=True,
                     out_scale=False)
